# baseline (speedup 1.0000x reference)
import functools

import jax
import jax.numpy as jnp
from jax import lax
from jax.experimental import pallas as pl
from jax.experimental.pallas import tpu as pltpu
from jax.experimental.pallas import tpu_sc as plsc

D_MODEL = 1024
SCALE = 32.0
NC, NS, L = 2, 16, 16
NW = NC * NS
B = 4 * 8192

BPW = 736
B_SC = BPW * NW
CH = 16
NCHUNK = BPW // CH
RING = 6
PD = 2
PEEL = RING - PD
VPR = D_MODEL // L

assert (NCHUNK - PEEL) % RING == 0

_mesh = plsc.VectorSubcoreMesh(core_axis_name="c", subcore_axis_name="s")


@functools.partial(
    pl.kernel,
    out_type=jax.ShapeDtypeStruct((B_SC, D_MODEL), jnp.float32),
    mesh=_mesh,
    scratch_types=[
        pltpu.VMEM((BPW,), jnp.int32),
    ] + [pltpu.VMEM((CH, D_MODEL), jnp.float32)] * RING
      + [pltpu.SemaphoreType.DMA] * (2 * RING),
)
def _embed_sc(x_hbm, table_hbm, out_hbm, idx_v, *rest):
    bufs = rest[:RING]
    gsems = rest[RING:2 * RING]
    ssems = rest[2 * RING:]

    wid = lax.axis_index("s") * NC + lax.axis_index("c")
    base = wid * BPW
    pltpu.sync_copy(x_hbm.at[pl.ds(base, BPW)], idx_v)

    def issue_gather(c, b):
        off = pl.multiple_of(c * CH, 8)
        pltpu.async_copy(table_hbm.at[idx_v.at[pl.ds(off, CH)]], bufs[b], gsems[b])

    def wait_gather(b):
        pltpu.make_async_copy(table_hbm.at[pl.ds(0, CH)], bufs[b], gsems[b]).wait()

    def scale_buf(b):
        buf = bufs[b]

        @plsc.parallel_loop(0, CH)
        def _(r):
            for j in range(VPR):
                buf[r, pl.ds(j * L, L)] = buf[r, pl.ds(j * L, L)] * SCALE

    def issue_scatter(c, b):
        off = pl.multiple_of(c * CH, 8)
        pltpu.async_copy(bufs[b], out_hbm.at[pl.ds(base + off, CH)], ssems[b])

    def wait_scatter(b):
        pltpu.make_async_copy(bufs[b], out_hbm.at[pl.ds(0, CH)], ssems[b]).wait()

    for c in range(PD):
        issue_gather(c, c)

    for c in range(PEEL):
        issue_gather(c + PD, c + PD)
        wait_gather(c)
        scale_buf(c)
        issue_scatter(c, c)

    def outer(t, carry):
        for i in range(RING):
            c = PEEL + t * RING + i
            b = (PEEL + i) % RING

            @pl.when(c + PD < NCHUNK)
            def _():
                nb = (b + PD) % RING
                wait_scatter(nb)
                issue_gather(c + PD, nb)

            wait_gather(b)
            scale_buf(b)
            issue_scatter(c, b)

        return carry

    lax.fori_loop(0, (NCHUNK - PEEL) // RING, outer, 0)

    for c in range(NCHUNK - RING, NCHUNK):
        wait_scatter(c % RING)


B_TC = B - B_SC
K = 8
SL = D_MODEL // 128

assert B_TC % K == 0


def _tc_in_map(k, i, idx_ref):
    return (idx_ref[i * K + k], 0, 0)


def _tc_body(idx_ref, *refs):
    in_refs = refs[:K]
    out_ref = refs[K]
    for k in range(K):
        out_ref[k] = in_refs[k][0] * SCALE


_tc_call = pl.pallas_call(
    _tc_body,
    grid_spec=pltpu.PrefetchScalarGridSpec(
        num_scalar_prefetch=1,
        grid=(B_TC // K,),
        in_specs=[
            pl.BlockSpec((1, SL, 128), functools.partial(_tc_in_map, k))
            for k in range(K)
        ],
        out_specs=pl.BlockSpec((K, SL, 128), lambda i, idx_ref: (i, 0, 0)),
    ),
    out_shape=jax.ShapeDtypeStruct((B_TC, SL, 128), jnp.float32),
)


def kernel(x, embedding):
    xf = x.reshape(-1).astype(jnp.int32)
    table3 = embedding.reshape(embedding.shape[0], SL, 128)
    out_sc = _embed_sc(xf[:B_SC], embedding)
    out_tc = _tc_call(xf[B_SC:], *([table3] * K))
    out = jnp.concatenate([out_sc, out_tc.reshape(B_TC, D_MODEL)], axis=0)
    return out.reshape(x.shape[0], x.shape[1], D_MODEL)


# speedup vs baseline: 9.0466x; 9.0466x over previous
import functools

import jax
import jax.numpy as jnp
from jax import lax
from jax.experimental import pallas as pl
from jax.experimental.pallas import tpu as pltpu
from jax.experimental.pallas import tpu_sc as plsc

D_MODEL = 1024
SCALE = 32.0
NC, NS, L = 2, 16, 16
NW = NC * NS
B = 4 * 8192
BPW = B // NW
CH = 32
NCHUNK = BPW // CH
RING = 3
VPR = D_MODEL // L

_mesh = plsc.VectorSubcoreMesh(core_axis_name="c", subcore_axis_name="s")


@functools.partial(
    pl.kernel,
    out_type=jax.ShapeDtypeStruct((B, D_MODEL), jnp.float32),
    mesh=_mesh,
    scratch_types=[
        pltpu.VMEM((BPW,), jnp.int32),
        pltpu.VMEM((CH, D_MODEL), jnp.float32),
        pltpu.VMEM((CH, D_MODEL), jnp.float32),
        pltpu.VMEM((CH, D_MODEL), jnp.float32),
        pltpu.SemaphoreType.DMA,
        pltpu.SemaphoreType.DMA,
        pltpu.SemaphoreType.DMA,
        pltpu.SemaphoreType.DMA,
        pltpu.SemaphoreType.DMA,
        pltpu.SemaphoreType.DMA,
    ],
)
def _embed_sc(x_hbm, table_hbm, out_hbm, idx_v,
              b0, b1, b2, g0, g1, g2, s0, s1, s2):
    wid = lax.axis_index("s") * NC + lax.axis_index("c")
    base = wid * BPW
    pltpu.sync_copy(x_hbm.at[pl.ds(base, BPW)], idx_v)

    bufs = (b0, b1, b2)
    gsems = (g0, g1, g2)
    ssems = (s0, s1, s2)

    def issue_gather(c, b):
        off = pl.multiple_of(c * CH, 8)
        pltpu.async_copy(table_hbm.at[idx_v.at[pl.ds(off, CH)]], bufs[b], gsems[b])

    def wait_gather(b):
        pltpu.make_async_copy(table_hbm.at[pl.ds(0, CH)], bufs[b], gsems[b]).wait()

    def scale_buf(b):
        buf = bufs[b]

        @plsc.parallel_loop(0, CH)
        def _(r):
            for j in range(VPR):
                buf[r, pl.ds(j * L, L)] = buf[r, pl.ds(j * L, L)] * SCALE

    def issue_scatter(c, b):
        off = pl.multiple_of(c * CH, 8)
        pltpu.async_copy(bufs[b], out_hbm.at[pl.ds(base + off, CH)], ssems[b])

    def wait_scatter(b):
        pltpu.make_async_copy(bufs[b], out_hbm.at[pl.ds(0, CH)], ssems[b]).wait()

    issue_gather(0, 0)
    issue_gather(1, 1)

    wait_gather(0)
    scale_buf(0)
    issue_scatter(0, 0)
    issue_gather(2, 2)

    wait_gather(1)
    scale_buf(1)
    issue_scatter(1, 1)
    wait_scatter(0)
    issue_gather(3, 0)

    def outer(t, carry):
        for b in range(RING):
            c = 2 + t * RING + b
            bb = (2 + b) % RING
            nb = (bb + 2) % RING

            wait_gather(bb)
            scale_buf(bb)
            issue_scatter(c, bb)

            @pl.when(c + 2 < NCHUNK)
            def _():
                wait_scatter(nb)
                issue_gather(c + 2, nb)

        return carry

    lax.fori_loop(0, (NCHUNK - 2) // RING, outer, 0)

    for b in range(RING):
        wait_scatter(b)


def kernel(x, embedding):
    xf = x.reshape(-1).astype(jnp.int32)
    out = _embed_sc(xf, embedding)
    return out.reshape(x.shape[0], x.shape[1], D_MODEL)
